# baseline (speedup 1.0000x reference)
import functools
import jax
import jax.numpy as jnp
from jax import lax
from jax.experimental import pallas as pl
from jax.experimental.pallas import tpu as pltpu
from jax.experimental.pallas import tpu_sc as plsc

BATCH = 16384
FIELDS = 26
DIM = 64
NC = 2
NS = 16
NW = NC * NS
BLK = BATCH // NW
BW = 128
CPB = BLK // BW
NTASK = FIELDS * CPB
NBUF = 4


def _make_emb():
  mesh = plsc.VectorSubcoreMesh(core_axis_name="c", subcore_axis_name="s")

  @functools.partial(
      pl.kernel,
      mesh=mesh,
      out_type=jax.ShapeDtypeStruct((FIELDS, BATCH, DIM), jnp.float32),
      scratch_types=[
          pltpu.VMEM((FIELDS * BLK,), jnp.int32),
          [pltpu.VMEM((BW, 2 * DIM), jnp.float32) for _ in range(NBUF)],
          [pltpu.VMEM((BW, DIM), jnp.float32) for _ in range(2)],
          [pltpu.SemaphoreType.DMA for _ in range(NBUF)],
          [pltpu.SemaphoreType.DMA for _ in range(2)],
          pltpu.SemaphoreType.DMA,
      ],
      compiler_params=pltpu.CompilerParams(needs_layout_passes=False),
  )
  def body(ids_hbm, w_hbm, out_hbm, idx_v, gb, cb, gsems, wsems, isem):
    wid = lax.axis_index("s") * NC + lax.axis_index("c")
    b0 = wid * BLK
    for f in range(FIELDS):
      pltpu.async_copy(
          ids_hbm.at[f, pl.ds(b0, BLK)], idx_v.at[pl.ds(f * BLK, BLK)], isem)
    pltpu.make_async_copy(
        ids_hbm.at[0, pl.ds(0, BLK)], idx_v.at[pl.ds(0, BLK)], isem).wait()

    def idx_slice(t):
      return idx_v.at[pl.ds((t // CPB) * BLK + (t % CPB) * BW, BW)]

    for p in range(NBUF):
      pltpu.async_copy(w_hbm.at[idx_slice(p)], gb[p], gsems[p])
    for f in range(FIELDS - 1):
      pltpu.make_async_copy(
          ids_hbm.at[0, pl.ds(0, BLK)], idx_v.at[pl.ds(0, BLK)], isem).wait()

    @pl.loop(0, NTASK, step=NBUF)
    def _(t0):
      for p in range(NBUF):
        t = t0 + p
        q = p % 2

        pltpu.make_async_copy(
            w_hbm.at[pl.ds(0, BW)], gb[p], gsems[p]).wait()

        @pl.when(t >= 2)
        def _():
          pltpu.make_async_copy(
              cb[q], out_hbm.at[0, pl.ds(0, BW), :], wsems[q]).wait()

        @plsc.parallel_loop(0, BW, unroll=8)
        def _(r):
          for m in range(DIM // 16):
            cb[q][r, pl.ds(m * 16, 16)] = gb[p][r, pl.ds(m * 16, 16)]

        @pl.when(t + NBUF < NTASK)
        def _():
          g = t + NBUF
          pltpu.async_copy(
              w_hbm.at[idx_v.at[pl.ds((g // CPB) * BLK + (g % CPB) * BW, BW)]],
              gb[p], gsems[p])

        f = t // CPB
        pltpu.async_copy(
            cb[q], out_hbm.at[f, pl.ds(b0 + (t % CPB) * BW, BW), :], wsems[q])

    for q in range(2):
      pltpu.make_async_copy(
          cb[q], out_hbm.at[0, pl.ds(0, BW), :], wsems[q]).wait()

  return body


def kernel(input_ids, weight):
  ids_t = input_ids.T.astype(jnp.int32)
  wp = jnp.pad(weight, ((0, 0), (0, DIM)))
  out = _make_emb()(ids_t, wp)
  return out.transpose(1, 0, 2)
